# baseline (speedup 1.0000x reference)
import functools

import jax
import jax.numpy as jnp
from jax.experimental import pallas as pl
from jax.experimental.pallas import tpu as pltpu

_NUM_SAMPLES = 1024
_NUM_SEG = 9
_FEAT = 3072
_NUM_CLASSES = 20

_DN = (((1,), (1,)), ((), ()))


def _dot_t(a, w):
    return jax.lax.dot_general(a, w, _DN, preferred_element_type=jnp.float32)


def _heads(xb, sf, wa_ref, ba_ref, wc_ref, bc_ref, wr_ref, br_ref):
    F = _FEAT
    xr = xb.reshape(-1, _NUM_SEG, F)
    start = (xr[:, 0, :] + xr[:, 1, :]) * (sf[:, 0:1] * 0.5)
    course = (xr[:, 2, :] + xr[:, 3, :] + xr[:, 4, :]
              + xr[:, 5, :] + xr[:, 6, :]) * 0.2
    end = (xr[:, 7, :] + xr[:, 8, :]) * (sf[:, 1:2] * 0.5)
    act = _dot_t(course, wa_ref[...]) + ba_ref[...]
    comp = (_dot_t(start, wc_ref[:, 0:F]) + _dot_t(course, wc_ref[:, F:2 * F])
            + _dot_t(end, wc_ref[:, 2 * F:3 * F]) + bc_ref[...])
    reg = (_dot_t(start, wr_ref[:, 0:F]) + _dot_t(course, wr_ref[:, F:2 * F])
           + _dot_t(end, wr_ref[:, 2 * F:3 * F]) + br_ref[...])
    return act, comp, reg


def _fused_kernel(xa_ref, xb_ref, sfa_ref, sfb_ref,
                  wa_ref, ba_ref, wc_ref, bc_ref, wr_ref, br_ref,
                  acta_ref, compa_ref, rega_ref,
                  actb_ref, compb_ref, regb_ref):
    a, c, r = _heads(xa_ref[...], sfa_ref[...], wa_ref, ba_ref, wc_ref, bc_ref,
                     wr_ref, br_ref)
    acta_ref[...], compa_ref[...], rega_ref[...] = a, c, r
    a, c, r = _heads(xb_ref[...], sfb_ref[...], wa_ref, ba_ref, wc_ref, bc_ref,
                     wr_ref, br_ref)
    actb_ref[...], compb_ref[...], regb_ref[...] = a, c, r


@functools.partial(jax.jit, static_argnames=("block",))
def _run(x, sf, W_act, b_act, W_comp, b_comp, W_reg, b_reg, block=64):
    grid = _NUM_SAMPLES // (2 * block)
    nw = lambda i: (0, 0)
    rows = block * _NUM_SEG
    H = _NUM_SAMPLES // 2
    outs = pl.pallas_call(
        _fused_kernel,
        grid=(grid,),
        in_specs=[
            pl.BlockSpec((rows, _FEAT), lambda i: (i, 0)),
            pl.BlockSpec((rows, _FEAT), lambda i, g=grid: (i + g, 0)),
            pl.BlockSpec((block, 2), lambda i: (i, 0)),
            pl.BlockSpec((block, 2), lambda i, g=grid: (i + g, 0)),
            pl.BlockSpec(W_act.shape, nw),
            pl.BlockSpec(b_act.shape, nw),
            pl.BlockSpec(W_comp.shape, nw),
            pl.BlockSpec(b_comp.shape, nw),
            pl.BlockSpec(W_reg.shape, nw),
            pl.BlockSpec(b_reg.shape, nw),
        ],
        out_specs=[
            pl.BlockSpec((block, _NUM_CLASSES + 1), lambda i: (i, 0)),
            pl.BlockSpec((block, _NUM_CLASSES), lambda i: (i, 0)),
            pl.BlockSpec((block, _NUM_CLASSES * 2), lambda i: (i, 0)),
            pl.BlockSpec((block, _NUM_CLASSES + 1), lambda i: (i, 0)),
            pl.BlockSpec((block, _NUM_CLASSES), lambda i: (i, 0)),
            pl.BlockSpec((block, _NUM_CLASSES * 2), lambda i: (i, 0)),
        ],
        out_shape=[
            jax.ShapeDtypeStruct((H, _NUM_CLASSES + 1), jnp.float32),
            jax.ShapeDtypeStruct((H, _NUM_CLASSES), jnp.float32),
            jax.ShapeDtypeStruct((H, _NUM_CLASSES * 2), jnp.float32),
            jax.ShapeDtypeStruct((H, _NUM_CLASSES + 1), jnp.float32),
            jax.ShapeDtypeStruct((H, _NUM_CLASSES), jnp.float32),
            jax.ShapeDtypeStruct((H, _NUM_CLASSES * 2), jnp.float32),
        ],
        compiler_params=pltpu.CompilerParams(
            dimension_semantics=("arbitrary",)),
    )(x, x, sf, sf, W_act, b_act, W_comp, b_comp, W_reg, b_reg)
    return outs


def kernel(x, scale_factors, W_act, b_act, W_comp, b_comp, W_reg, b_reg):
    aa, ca, ra, ab, cb, rb = _run(x, scale_factors,
                                  W_act, b_act.reshape(1, -1),
                                  W_comp, b_comp.reshape(1, -1),
                                  W_reg, b_reg.reshape(1, -1))
    act = jnp.concatenate([aa, ab], axis=0)
    comp = jnp.concatenate([ca, cb], axis=0)
    reg = jnp.concatenate([ra, rb], axis=0)
    return (act, comp, reg.reshape(-1, _NUM_CLASSES, 2))
